# baseline (speedup 1.0000x reference)
import jax
import jax.numpy as jnp
from jax.experimental import pallas as pl


def _body(row_ref, col_ref, out_ref):
    h = row_ref.shape[0]
    out_n = row_ref.shape[1]
    m = h * h

    m_ids = jax.lax.broadcasted_iota(jnp.int32, (h, m), 1)
    r_ids = jax.lax.broadcasted_iota(jnp.int32, (h, m), 0)
    sel_q = (m_ids % h == r_ids).astype(jnp.float32)
    sel_p = (m_ids // h == r_ids).astype(jnp.float32)

    dn = (((0,), (0,)), ((), ()))
    top = jax.lax.dot_general(row_ref[...], sel_q, dn,
                              precision=jax.lax.Precision.HIGHEST)
    bot = jax.lax.dot_general(col_ref[...], sel_p, dn,
                              precision=jax.lax.Precision.HIGHEST)
    slab = jnp.concatenate([top, bot], axis=0)
    out_ref[...] = jnp.broadcast_to(slab[None], out_ref.shape)


def kernel(x, row_table, col_table):
    bs, _, h, w = x.shape
    out_n = row_table.shape[1]
    c_total = 2 * out_n
    m = h * w
    bblk = 2

    flat = pl.pallas_call(
        _body,
        grid=(bs // bblk,),
        in_specs=[
            pl.BlockSpec((h, out_n), lambda b: (0, 0)),
            pl.BlockSpec((w, out_n), lambda b: (0, 0)),
        ],
        out_specs=pl.BlockSpec((bblk, c_total, m), lambda b: (b, 0, 0)),
        out_shape=jax.ShapeDtypeStruct((bs, c_total, m), jnp.float32),
    )(row_table[:h], col_table[:w])
    return flat.reshape(bs, c_total, h, w)


# speedup vs baseline: 1.1173x; 1.1173x over previous
import jax
import jax.numpy as jnp
from jax.experimental import pallas as pl
from jax.experimental.pallas import tpu as pltpu


def _body(row_ref, col_ref, out_ref, slab_ref, sem):
    h = row_ref.shape[0]
    m = h * h

    m_ids = jax.lax.broadcasted_iota(jnp.int32, (h, m), 1)
    r_ids = jax.lax.broadcasted_iota(jnp.int32, (h, m), 0)
    sel_q = (m_ids % h == r_ids).astype(jnp.float32)
    sel_p = (m_ids // h == r_ids).astype(jnp.float32)

    dn = (((0,), (0,)), ((), ()))
    top = jax.lax.dot_general(row_ref[...], sel_q, dn,
                              precision=jax.lax.Precision.HIGHEST)
    bot = jax.lax.dot_general(col_ref[...], sel_p, dn,
                              precision=jax.lax.Precision.HIGHEST)
    slab_ref[...] = jnp.concatenate([top, bot], axis=0)

    bs = out_ref.shape[0]
    copies = [
        pltpu.make_async_copy(slab_ref, out_ref.at[b], sem)
        for b in range(bs)
    ]
    for c in copies:
        c.start()
    for c in copies:
        c.wait()


def kernel(x, row_table, col_table):
    bs, _, h, w = x.shape
    out_n = row_table.shape[1]
    c_total = 2 * out_n
    m = h * w

    flat = pl.pallas_call(
        _body,
        in_specs=[
            pl.BlockSpec(memory_space=pltpu.VMEM),
            pl.BlockSpec(memory_space=pltpu.VMEM),
        ],
        out_specs=pl.BlockSpec(memory_space=pl.ANY),
        out_shape=jax.ShapeDtypeStruct((bs, c_total, m), jnp.float32),
        scratch_shapes=[
            pltpu.VMEM((c_total, m), jnp.float32),
            pltpu.SemaphoreType.DMA,
        ],
    )(row_table[:h], col_table[:w])
    return flat.reshape(bs, c_total, h, w)


# speedup vs baseline: 1.1238x; 1.0058x over previous
import jax
import jax.numpy as jnp
from jax.experimental import pallas as pl
from jax.experimental.pallas import tpu as pltpu


def _body(row_ref, col_ref, out_ref, slab_ref):
    h = row_ref.shape[0]
    m = h * h

    @pl.when(pl.program_id(0) == 0)
    def _():
        m_ids = jax.lax.broadcasted_iota(jnp.int32, (h, m), 1)
        r_ids = jax.lax.broadcasted_iota(jnp.int32, (h, m), 0)
        sel_q = (m_ids % h == r_ids).astype(jnp.float32)
        sel_p = (m_ids // h == r_ids).astype(jnp.float32)
        dn = (((0,), (0,)), ((), ()))
        top = jax.lax.dot_general(row_ref[...], sel_q, dn,
                                  precision=jax.lax.Precision.HIGHEST)
        bot = jax.lax.dot_general(col_ref[...], sel_p, dn,
                                  precision=jax.lax.Precision.HIGHEST)
        slab_ref[...] = jnp.concatenate([top, bot], axis=0)

    out_ref[...] = jnp.broadcast_to(slab_ref[...][None], out_ref.shape)


def kernel(x, row_table, col_table):
    bs, _, h, w = x.shape
    out_n = row_table.shape[1]
    c_total = 2 * out_n
    m = h * w
    bblk = 4

    flat = pl.pallas_call(
        _body,
        grid=(bs // bblk,),
        in_specs=[
            pl.BlockSpec((h, out_n), lambda b: (0, 0)),
            pl.BlockSpec((w, out_n), lambda b: (0, 0)),
        ],
        out_specs=pl.BlockSpec((bblk, c_total, m), lambda b: (b, 0, 0)),
        out_shape=jax.ShapeDtypeStruct((bs, c_total, m), jnp.float32),
        scratch_shapes=[pltpu.VMEM((c_total, m), jnp.float32)],
    )(row_table[:h], col_table[:w])
    return flat.reshape(bs, c_total, h, w)
